# baseline (speedup 1.0000x reference)
import functools

import jax
import jax.numpy as jnp
from jax import lax
from jax.experimental import pallas as pl
from jax.experimental.pallas import tpu as pltpu
from jax.experimental.pallas import tpu_sc as plsc

B, N, D = 8, 576, 64
NTOK = B * N
K = 8192
TM = 576
GRID = NTOK // TM

NC, NS = 2, 16
NW = NC * NS
BPW = NTOK // NW
HALF = BPW // 2


def _argmin_body(z_ref, embT2_ref, idx_ref, dsum_ref, enorm_ref):
    step = pl.program_id(0)

    @pl.when(step == 0)
    def _():
        e2 = embT2_ref[...]
        enorm_ref[...] = 0.25 * jnp.sum(e2 * e2, axis=0, keepdims=True)

    z = z_ref[...].reshape(TM, D)
    dot2 = jnp.dot(z, embT2_ref[...], preferred_element_type=jnp.float32)
    znorm = jnp.sum(z * z, axis=1, keepdims=True)
    d = (znorm + enorm_ref[...]) - dot2
    m = jnp.min(d, axis=1, keepdims=True)
    idx_ref[0, 0, :] = jnp.argmin(d, axis=1).astype(jnp.int32)

    part = jnp.sum(m, keepdims=True).reshape(1, 1)
    prev = jnp.where(step == 0, jnp.zeros((1, 1), jnp.float32), dsum_ref[...])
    dsum_ref[...] = prev + part


_argmin_call = pl.pallas_call(
    _argmin_body,
    grid=(GRID,),
    in_specs=[
        pl.BlockSpec((1, TM, D), lambda i: (i, 0, 0)),
        pl.BlockSpec((D, K), lambda i: (0, 0)),
    ],
    out_specs=[
        pl.BlockSpec((1, 1, TM), lambda i: (i, 0, 0)),
        pl.BlockSpec((1, 1), lambda i: (0, 0)),
    ],
    out_shape=[
        jax.ShapeDtypeStruct((GRID, 1, TM), jnp.int32),
        jax.ShapeDtypeStruct((1, 1), jnp.float32),
    ],
    scratch_shapes=[pltpu.VMEM((1, K), jnp.float32)],
)


WPR = N // BPW


def _gather_body(table_hbm, idx_hbm, out_hbm, idx_v, rows_v, sem):
    wid = lax.axis_index("s") * NC + lax.axis_index("c")
    b = wid // WPR
    n0 = (wid % WPR) * BPW
    pltpu.sync_copy(idx_hbm.at[pl.ds(wid * BPW, BPW)], idx_v)
    c0 = pltpu.async_copy(table_hbm.at[idx_v.at[pl.ds(0, HALF)]],
                          rows_v.at[pl.ds(0, HALF)], sem)
    c1 = pltpu.async_copy(table_hbm.at[idx_v.at[pl.ds(HALF, HALF)]],
                          rows_v.at[pl.ds(HALF, HALF)], sem)
    c0.wait()
    c1.wait()
    pltpu.sync_copy(rows_v, out_hbm.at[b, pl.ds(n0, BPW)])


@functools.cache
def _gather_call():
    return pl.kernel(
        _gather_body,
        out_type=jax.ShapeDtypeStruct((B, N, D), jnp.float32),
        mesh=plsc.VectorSubcoreMesh(core_axis_name="c", subcore_axis_name="s"),
        scratch_types=[
            pltpu.VMEM((BPW,), jnp.int32),
            pltpu.VMEM((BPW, D), jnp.float32),
            pltpu.SemaphoreType.DMA,
        ],
        compiler_params=pltpu.CompilerParams(use_tc_tiling_on_sc=False),
    )


def kernel(z, emb_weight):
    z = z.astype(jnp.float32)
    idx3, dsum = _argmin_call(z, (emb_weight * 2.0).T)
    q_indices = idx3.reshape(B, N)
    quantized = _gather_call()(emb_weight, idx3.reshape(NTOK))
    mse = dsum.reshape(()) / float(NTOK * D)
    loss = 1.25 * mse
    zero = jnp.zeros((), jnp.float32)
    return (z, emb_weight, quantized, q_indices, loss, mse, mse,
            zero, zero, zero)


# speedup vs baseline: 1.0470x; 1.0470x over previous
import functools

import jax
import jax.numpy as jnp
from jax import lax
from jax.experimental import pallas as pl
from jax.experimental.pallas import tpu as pltpu
from jax.experimental.pallas import tpu_sc as plsc

B, N, D = 8, 576, 64
NTOK = B * N
K = 8192
TM = 576
GRID = NTOK // TM

NC, NS = 2, 16
NW = NC * NS
BPW = NTOK // NW
HALF = BPW // 2


def _argmin_body(z_ref, embT2_ref, idx_ref, enorm_ref):
    step = pl.program_id(0)

    @pl.when(step == 0)
    def _():
        e2 = embT2_ref[...]
        enorm_ref[...] = 0.25 * jnp.sum(e2 * e2, axis=0, keepdims=True)

    z = z_ref[...].reshape(TM, D)
    dot2 = jnp.dot(z, embT2_ref[...], preferred_element_type=jnp.float32)
    znorm = jnp.sum(z * z, axis=1, keepdims=True)
    d = (znorm + enorm_ref[...]) - dot2
    idx = jnp.argmin(d, axis=1).astype(jnp.int32)
    idx_ref[step, :] = idx


_argmin_call = pl.pallas_call(
    _argmin_body,
    grid=(GRID,),
    in_specs=[
        pl.BlockSpec((1, TM, D), lambda i: (i, 0, 0)),
        pl.BlockSpec((D, K), lambda i: (0, 0)),
    ],
    out_specs=pl.BlockSpec((GRID, TM), lambda i: (0, 0)),
    out_shape=jax.ShapeDtypeStruct((GRID, TM), jnp.int32),
    scratch_shapes=[pltpu.VMEM((1, K), jnp.float32)],
)


WPR = N // BPW


def _gather_body(table_hbm, idx_hbm, z_hbm, out_hbm, part_hbm,
                 idx_v, rows_v, z_v, acc_v, sem):
    wid = lax.axis_index("s") * NC + lax.axis_index("c")
    b = wid // WPR
    n0 = (wid % WPR) * BPW
    pltpu.sync_copy(idx_hbm.at[b, pl.ds(n0, BPW)], idx_v)
    c0 = pltpu.async_copy(table_hbm.at[idx_v.at[pl.ds(0, HALF)]],
                          rows_v.at[pl.ds(0, HALF)], sem)
    c1 = pltpu.async_copy(table_hbm.at[idx_v.at[pl.ds(HALF, HALF)]],
                          rows_v.at[pl.ds(HALF, HALF)], sem)
    pltpu.sync_copy(z_hbm.at[b, pl.ds(n0, BPW)], z_v)
    c0.wait()
    c1.wait()
    pltpu.sync_copy(rows_v, out_hbm.at[b, pl.ds(n0, BPW)])

    def _row(r, acc):
        for c in range(D // 16):
            q16 = rows_v[r, pl.ds(c * 16, 16)]
            z16 = z_v[r, pl.ds(c * 16, 16)]
            t = q16 - z16
            acc = acc + t * t
        return acc

    acc_v[...] = lax.fori_loop(0, BPW, _row, jnp.zeros((16,), jnp.float32))
    pltpu.sync_copy(acc_v, part_hbm.at[wid])


@functools.cache
def _gather_call():
    return pl.kernel(
        _gather_body,
        out_type=[
            jax.ShapeDtypeStruct((B, N, D), jnp.float32),
            jax.ShapeDtypeStruct((NW, 16), jnp.float32),
        ],
        mesh=plsc.VectorSubcoreMesh(core_axis_name="c", subcore_axis_name="s"),
        scratch_types=[
            pltpu.VMEM((BPW,), jnp.int32),
            pltpu.VMEM((BPW, D), jnp.float32),
            pltpu.VMEM((BPW, D), jnp.float32),
            pltpu.VMEM((16,), jnp.float32),
            pltpu.SemaphoreType.DMA,
        ],
        compiler_params=pltpu.CompilerParams(use_tc_tiling_on_sc=False),
    )


def kernel(z, emb_weight):
    z = z.astype(jnp.float32)
    q_indices = _argmin_call(z, (emb_weight * 2.0).T)
    quantized, parts = _gather_call()(emb_weight, q_indices, z)
    mse = jnp.sum(parts) / float(NTOK * D)
    loss = 1.25 * mse
    zero = jnp.zeros((), jnp.float32)
    return (z, emb_weight, quantized, q_indices, loss, mse, mse,
            zero, zero, zero)


# speedup vs baseline: 1.0951x; 1.0459x over previous
import functools

import jax
import jax.numpy as jnp
from jax import lax
from jax.experimental import pallas as pl
from jax.experimental.pallas import tpu as pltpu
from jax.experimental.pallas import tpu_sc as plsc

B, N, D = 8, 576, 64
NTOK = B * N
K = 8192
TM = 576
GRID = NTOK // TM

NC, NS = 2, 16
NW = NC * NS
BPW = NTOK // NW
HALF = BPW // 2


def _argmin_body(z_ref, emb_ref, idx_ref, embT2_ref, enorm_ref):
    step = pl.program_id(0)

    @pl.when(step == 0)
    def _():
        e2 = 2.0 * jnp.transpose(emb_ref[...])
        embT2_ref[...] = e2
        enorm_ref[...] = 0.25 * jnp.sum(e2 * e2, axis=0, keepdims=True)

    z = z_ref[...].reshape(TM, D)
    dot2 = jnp.dot(z, embT2_ref[...], preferred_element_type=jnp.float32)
    znorm = jnp.sum(z * z, axis=1, keepdims=True)
    d = (znorm + enorm_ref[...]) - dot2
    idx = jnp.argmin(d, axis=1).astype(jnp.int32)
    idx_ref[step, :] = idx


_argmin_call = pl.pallas_call(
    _argmin_body,
    grid=(GRID,),
    in_specs=[
        pl.BlockSpec((1, TM, D), lambda i: (i, 0, 0)),
        pl.BlockSpec((K, D), lambda i: (0, 0)),
    ],
    out_specs=pl.BlockSpec((GRID, TM), lambda i: (0, 0)),
    out_shape=jax.ShapeDtypeStruct((GRID, TM), jnp.int32),
    scratch_shapes=[pltpu.VMEM((D, K), jnp.float32),
                    pltpu.VMEM((1, K), jnp.float32)],
)


WPR = N // BPW


def _gather_body(table_hbm, idx_hbm, z_hbm, out_hbm, part_hbm,
                 idx_v, rows_v, z_v, acc_v, sem):
    wid = lax.axis_index("s") * NC + lax.axis_index("c")
    b = wid // WPR
    n0 = (wid % WPR) * BPW
    pltpu.sync_copy(idx_hbm.at[b, pl.ds(n0, BPW)], idx_v)
    c0 = pltpu.async_copy(table_hbm.at[idx_v.at[pl.ds(0, HALF)]],
                          rows_v.at[pl.ds(0, HALF)], sem)
    c1 = pltpu.async_copy(table_hbm.at[idx_v.at[pl.ds(HALF, HALF)]],
                          rows_v.at[pl.ds(HALF, HALF)], sem)
    pltpu.sync_copy(z_hbm.at[b, pl.ds(n0, BPW)], z_v)
    c0.wait()
    c1.wait()
    pltpu.sync_copy(rows_v, out_hbm.at[b, pl.ds(n0, BPW)])

    def _row(r, acc):
        for c in range(D // 16):
            q16 = rows_v[r, pl.ds(c * 16, 16)]
            z16 = z_v[r, pl.ds(c * 16, 16)]
            t = q16 - z16
            acc = acc + t * t
        return acc

    acc_v[...] = lax.fori_loop(0, BPW, _row, jnp.zeros((16,), jnp.float32))
    pltpu.sync_copy(acc_v, part_hbm.at[wid])


@functools.cache
def _gather_call():
    return pl.kernel(
        _gather_body,
        out_type=[
            jax.ShapeDtypeStruct((B, N, D), jnp.float32),
            jax.ShapeDtypeStruct((NW, 16), jnp.float32),
        ],
        mesh=plsc.VectorSubcoreMesh(core_axis_name="c", subcore_axis_name="s"),
        scratch_types=[
            pltpu.VMEM((BPW,), jnp.int32),
            pltpu.VMEM((BPW, D), jnp.float32),
            pltpu.VMEM((BPW, D), jnp.float32),
            pltpu.VMEM((16,), jnp.float32),
            pltpu.SemaphoreType.DMA,
        ],
        compiler_params=pltpu.CompilerParams(use_tc_tiling_on_sc=False),
    )


def kernel(z, emb_weight):
    z = z.astype(jnp.float32)
    q_indices = _argmin_call(z, emb_weight)
    quantized, parts = _gather_call()(emb_weight, q_indices, z)
    mse = jnp.sum(parts) / float(NTOK * D)
    loss = 1.25 * mse
    zero = jnp.zeros((), jnp.float32)
    return (z, emb_weight, quantized, q_indices, loss, mse, mse,
            zero, zero, zero)
